# Initial kernel scaffold; baseline (speedup 1.0000x reference)
import jax
import jax.numpy as jnp
from jax.experimental import pallas as pl


def kernel(X, Wm, Ws, M):
    raise NotImplementedError("write your pallas kernel here")


# speedup vs baseline: 2.8991x; 2.8991x over previous
import functools

import jax
import jax.numpy as jnp
from jax.experimental import pallas as pl
from jax.experimental.pallas import tpu as pltpu

N = 196
C = 384
C2 = 192
K = 8
NP = 224
BI = 16
NIB = NP // BI


def _srknn_body(xv_blk, xv_full, wm, ws, m, idx_out, val_out, vt_s, a_s):
    b = pl.program_id(0)
    i = pl.program_id(1)

    @pl.when(jnp.logical_and(b == 0, i == 0))
    def _():
        a_s[...] = jax.lax.dot_general(
            m[...], m[...], (((0,), (0,)), ((), ())),
            preferred_element_type=jnp.float32)

    @pl.when(i == 0)
    def _():
        vt_s[...] = jax.lax.dot_general(
            ws[...], xv_full[0], (((1,), (1,)), ((), ())),
            preferred_element_type=jnp.float32)

    ut = jax.lax.dot_general(
        wm[...], xv_blk[0], (((1,), (1,)), ((), ())),
        preferred_element_type=jnp.float32)

    vt = vt_s[...]
    a = a_s[...]

    rows = []
    for r in range(BI):
        s = jax.nn.sigmoid(ut[:, r:r + 1] - vt)
        t = jax.lax.dot_general(
            a, s, (((1,), (0,)), ((), ())),
            preferred_element_type=jnp.float32)
        d2 = jnp.sum(t * s, axis=0, keepdims=True)
        rows.append(d2)
    d2b = jnp.maximum(jnp.concatenate(rows, axis=0), 0.0)
    diff = jax.nn.sigmoid(-jnp.sqrt(d2b))

    lane = jax.lax.broadcasted_iota(jnp.int32, (BI, NP), 1)
    diff = jnp.where(lane < N, diff, -jnp.inf)

    vals, idxs = [], []
    for _ in range(K):
        mx = jnp.max(diff, axis=1, keepdims=True)
        amx = jnp.min(jnp.where(diff == mx, lane, NP), axis=1,
                      keepdims=True)
        vals.append(-mx)
        idxs.append(amx)
        diff = jnp.where(lane == amx, -jnp.inf, diff)
    val_out[0] = jnp.concatenate(vals, axis=1)
    idx_out[0] = jnp.concatenate(idxs, axis=1)


@jax.jit
def kernel(X, Wm, Ws, M):
    B = X.shape[0]
    XV = jnp.transpose(X, (0, 2, 3, 1)).reshape(B, N, C)
    XVp = jnp.pad(XV, ((0, 0), (0, NP - N), (0, 0)))

    grid = (B, NIB)
    idx_pad, val_pad = pl.pallas_call(
        _srknn_body,
        grid=grid,
        in_specs=[
            pl.BlockSpec((1, BI, C), lambda b, i: (b, i, 0)),
            pl.BlockSpec((1, NP, C), lambda b, i: (b, 0, 0)),
            pl.BlockSpec((C2, C), lambda b, i: (0, 0)),
            pl.BlockSpec((C2, C), lambda b, i: (0, 0)),
            pl.BlockSpec((C2, C2), lambda b, i: (0, 0)),
        ],
        out_specs=[
            pl.BlockSpec((1, BI, K), lambda b, i: (b, i, 0)),
            pl.BlockSpec((1, BI, K), lambda b, i: (b, i, 0)),
        ],
        out_shape=[
            jax.ShapeDtypeStruct((B, NP, K), jnp.int32),
            jax.ShapeDtypeStruct((B, NP, K), jnp.float32),
        ],
        scratch_shapes=[
            pltpu.VMEM((C2, NP), jnp.float32),
            pltpu.VMEM((C2, C2), jnp.float32),
        ],
    )(XVp, XVp, Wm, Ws, M)

    index = idx_pad[:, :N, :].reshape(B, N * K)
    value = val_pad[:, :N, :].reshape(B, N * K, 1)
    return (index, value)


# speedup vs baseline: 2.9499x; 1.0175x over previous
import functools

import jax
import jax.numpy as jnp
from jax.experimental import pallas as pl
from jax.experimental.pallas import tpu as pltpu

N = 196
C = 384
C2 = 192
K = 8
NP = 224
NPJ = 256
BI = 16
NIB = NP // BI


def _srknn_body(xv_blk, xv_full, wm, ws, m, idx_out, val_out, vt_s, a_s):
    b = pl.program_id(0)
    i = pl.program_id(1)

    @pl.when(jnp.logical_and(b == 0, i == 0))
    def _():
        a_s[...] = jax.lax.dot_general(
            m[...], m[...], (((0,), (0,)), ((), ())),
            preferred_element_type=jnp.float32)

    @pl.when(i == 0)
    def _():
        vt_s[...] = jax.lax.dot_general(
            ws[...], xv_full[0], (((1,), (1,)), ((), ())),
            preferred_element_type=jnp.float32)

    ut = jax.lax.dot_general(
        wm[...], xv_blk[0], (((1,), (1,)), ((), ())),
        preferred_element_type=jnp.float32)

    vt = vt_s[...]
    a = a_s[...]

    s_cat = jnp.concatenate(
        [jax.nn.sigmoid(ut[:, r:r + 1] - vt) for r in range(BI)],
        axis=1)
    t_cat = jax.lax.dot_general(
        a, s_cat, (((1,), (0,)), ((), ())),
        preferred_element_type=jnp.float32)
    p_cat = t_cat * s_cat
    rows = [jnp.sum(p_cat[:, r * NPJ:(r + 1) * NPJ], axis=0, keepdims=True)
            for r in range(BI)]
    d2b = jnp.maximum(jnp.concatenate(rows, axis=0), 0.0)
    diff = jax.nn.sigmoid(-jnp.sqrt(d2b))

    lane = jax.lax.broadcasted_iota(jnp.int32, (BI, NPJ), 1)
    diff = jnp.where(lane < N, diff, -jnp.inf)

    vals, idxs = [], []
    for _ in range(K):
        mx = jnp.max(diff, axis=1, keepdims=True)
        amx = jnp.min(jnp.where(diff == mx, lane, NP), axis=1,
                      keepdims=True)
        vals.append(-mx)
        idxs.append(amx)
        diff = jnp.where(lane == amx, -jnp.inf, diff)
    val_out[0] = jnp.concatenate(vals, axis=1)
    idx_out[0] = jnp.concatenate(idxs, axis=1)


@jax.jit
def kernel(X, Wm, Ws, M):
    B = X.shape[0]
    XV = jnp.transpose(X, (0, 2, 3, 1)).reshape(B, N, C)
    XVp = jnp.pad(XV, ((0, 0), (0, NPJ - N), (0, 0)))

    grid = (B, NIB)
    idx_pad, val_pad = pl.pallas_call(
        _srknn_body,
        grid=grid,
        in_specs=[
            pl.BlockSpec((1, BI, C), lambda b, i: (b, i, 0)),
            pl.BlockSpec((1, NPJ, C), lambda b, i: (b, 0, 0)),
            pl.BlockSpec((C2, C), lambda b, i: (0, 0)),
            pl.BlockSpec((C2, C), lambda b, i: (0, 0)),
            pl.BlockSpec((C2, C2), lambda b, i: (0, 0)),
        ],
        out_specs=[
            pl.BlockSpec((1, BI, K), lambda b, i: (b, i, 0)),
            pl.BlockSpec((1, BI, K), lambda b, i: (b, i, 0)),
        ],
        out_shape=[
            jax.ShapeDtypeStruct((B, NP, K), jnp.int32),
            jax.ShapeDtypeStruct((B, NP, K), jnp.float32),
        ],
        scratch_shapes=[
            pltpu.VMEM((C2, NPJ), jnp.float32),
            pltpu.VMEM((C2, C2), jnp.float32),
        ],
    )(XVp, XVp, Wm, Ws, M)

    index = idx_pad[:, :N, :].reshape(B, N * K)
    value = val_pad[:, :N, :].reshape(B, N * K, 1)
    return (index, value)


# speedup vs baseline: 5.4083x; 1.8334x over previous
import functools

import jax
import jax.numpy as jnp
from jax.experimental import pallas as pl
from jax.experimental.pallas import tpu as pltpu

N = 196
C = 384
C2 = 192
K = 8
NP = 224
NPJ = 256
BI = 16
NIB = NP // BI


def _dist_body(xv_blk, xv_full, wm, ws, m, diff_out, vt_s, a_s):
    b = pl.program_id(0)
    i = pl.program_id(1)

    @pl.when(jnp.logical_and(b == 0, i == 0))
    def _():
        a_s[...] = jax.lax.dot_general(
            m[...], m[...], (((0,), (0,)), ((), ())),
            preferred_element_type=jnp.float32)

    @pl.when(i == 0)
    def _():
        vt_s[...] = jax.lax.dot_general(
            ws[...], xv_full[0], (((1,), (1,)), ((), ())),
            preferred_element_type=jnp.float32)

    ut = jax.lax.dot_general(
        wm[...], xv_blk[0], (((1,), (1,)), ((), ())),
        preferred_element_type=jnp.float32)

    vt = vt_s[...]
    a = a_s[...]

    s_cat = jnp.concatenate(
        [jax.nn.sigmoid(ut[:, r:r + 1] - vt) for r in range(BI)],
        axis=1)
    t_cat = jax.lax.dot_general(
        a, s_cat, (((1,), (0,)), ((), ())),
        preferred_element_type=jnp.float32)
    p_cat = t_cat * s_cat
    rows = [jnp.sum(p_cat[:, r * NPJ:(r + 1) * NPJ], axis=0, keepdims=True)
            for r in range(BI)]
    d2b = jnp.maximum(jnp.concatenate(rows, axis=0), 0.0)
    diff = jax.nn.sigmoid(-jnp.sqrt(d2b))

    lane = jax.lax.broadcasted_iota(jnp.int32, (BI, NPJ), 1)
    diff_out[0] = jnp.where(lane < N, diff, -jnp.inf)


def _topk_body(diff_ref, idx_out, val_out):
    diff = diff_ref[...]
    R = diff.shape[0]
    lane = jax.lax.broadcasted_iota(jnp.int32, (R, NPJ), 1)
    vals, idxs = [], []
    for _ in range(K):
        mx = jnp.max(diff, axis=1, keepdims=True)
        amx = jnp.min(jnp.where(diff == mx, lane, NPJ), axis=1,
                      keepdims=True)
        vals.append(-mx)
        idxs.append(amx)
        diff = jnp.where(lane == amx, -jnp.inf, diff)
    val_out[...] = jnp.concatenate(vals, axis=1)
    idx_out[...] = jnp.concatenate(idxs, axis=1)


@jax.jit
def kernel(X, Wm, Ws, M):
    B = X.shape[0]
    XV = jnp.transpose(X, (0, 2, 3, 1)).reshape(B, N, C)
    XVp = jnp.pad(XV, ((0, 0), (0, NPJ - N), (0, 0)))

    diff = pl.pallas_call(
        _dist_body,
        grid=(B, NIB),
        in_specs=[
            pl.BlockSpec((1, BI, C), lambda b, i: (b, i, 0)),
            pl.BlockSpec((1, NPJ, C), lambda b, i: (b, 0, 0)),
            pl.BlockSpec((C2, C), lambda b, i: (0, 0)),
            pl.BlockSpec((C2, C), lambda b, i: (0, 0)),
            pl.BlockSpec((C2, C2), lambda b, i: (0, 0)),
        ],
        out_specs=pl.BlockSpec((1, BI, NPJ), lambda b, i: (b, i, 0)),
        out_shape=jax.ShapeDtypeStruct((B, NP, NPJ), jnp.float32),
        scratch_shapes=[
            pltpu.VMEM((C2, NPJ), jnp.float32),
            pltpu.VMEM((C2, C2), jnp.float32),
        ],
    )(XVp, XVp, Wm, Ws, M)

    R = B * NP
    idx_pad, val_pad = pl.pallas_call(
        _topk_body,
        out_shape=[
            jax.ShapeDtypeStruct((R, K), jnp.int32),
            jax.ShapeDtypeStruct((R, K), jnp.float32),
        ],
    )(diff.reshape(R, NPJ))

    index = idx_pad.reshape(B, NP, K)[:, :N, :].reshape(B, N * K)
    value = val_pad.reshape(B, NP, K)[:, :N, :].reshape(B, N * K, 1)
    return (index, value)


# speedup vs baseline: 6.6442x; 1.2285x over previous
import functools

import jax
import jax.numpy as jnp
from jax.experimental import pallas as pl
from jax.experimental.pallas import tpu as pltpu

N = 196
C = 384
C2 = 192
K = 8
NP = 224
NPJ = 256
BI = 16
NIB = NP // BI


def _dist_body(xv_blk, xv_full, wm, ws, m, diff_out, vt_s, a_s):
    b = pl.program_id(0)
    i = pl.program_id(1)

    @pl.when(jnp.logical_and(b == 0, i == 0))
    def _():
        a_s[...] = jax.lax.dot_general(
            m[...], m[...], (((0,), (0,)), ((), ())),
            preferred_element_type=jnp.float32)

    @pl.when(i == 0)
    def _():
        vt_s[...] = jnp.exp(jax.lax.dot_general(
            ws[...], xv_full[0], (((1,), (1,)), ((), ())),
            preferred_element_type=jnp.float32))

    ft = jnp.exp(-jax.lax.dot_general(
        wm[...], xv_blk[0], (((1,), (1,)), ((), ())),
        preferred_element_type=jnp.float32))

    ev = vt_s[...]
    a = a_s[...]

    s_cat = jnp.concatenate(
        [1.0 / (1.0 + ft[:, r:r + 1] * ev) for r in range(BI)],
        axis=1)
    t_cat = jax.lax.dot_general(
        a, s_cat, (((1,), (0,)), ((), ())),
        preferred_element_type=jnp.float32)
    p_cat = t_cat * s_cat
    rows = [jnp.sum(p_cat[:, r * NPJ:(r + 1) * NPJ], axis=0, keepdims=True)
            for r in range(BI)]
    d2b = jnp.maximum(jnp.concatenate(rows, axis=0), 0.0)
    diff = jax.nn.sigmoid(-jnp.sqrt(d2b))

    lane = jax.lax.broadcasted_iota(jnp.int32, (BI, NPJ), 1)
    diff_out[0] = jnp.where(lane < N, diff, -jnp.inf)


def _topk_body(diff_ref, idx_out, val_out):
    diff = diff_ref[...]
    R = diff.shape[0]
    lane = jax.lax.broadcasted_iota(jnp.int32, (R, NPJ), 1)
    vals, idxs = [], []
    for _ in range(K):
        mx = jnp.max(diff, axis=1, keepdims=True)
        amx = jnp.min(jnp.where(diff == mx, lane, NPJ), axis=1,
                      keepdims=True)
        vals.append(-mx)
        idxs.append(amx)
        diff = jnp.where(lane == amx, -jnp.inf, diff)
    val_out[...] = jnp.concatenate(vals, axis=1)
    idx_out[...] = jnp.concatenate(idxs, axis=1)


@jax.jit
def kernel(X, Wm, Ws, M):
    B = X.shape[0]
    XV = jnp.transpose(X, (0, 2, 3, 1)).reshape(B, N, C)
    XVp = jnp.pad(XV, ((0, 0), (0, NPJ - N), (0, 0)))

    diff = pl.pallas_call(
        _dist_body,
        grid=(B, NIB),
        in_specs=[
            pl.BlockSpec((1, BI, C), lambda b, i: (b, i, 0)),
            pl.BlockSpec((1, NPJ, C), lambda b, i: (b, 0, 0)),
            pl.BlockSpec((C2, C), lambda b, i: (0, 0)),
            pl.BlockSpec((C2, C), lambda b, i: (0, 0)),
            pl.BlockSpec((C2, C2), lambda b, i: (0, 0)),
        ],
        out_specs=pl.BlockSpec((1, BI, NPJ), lambda b, i: (b, i, 0)),
        out_shape=jax.ShapeDtypeStruct((B, NP, NPJ), jnp.float32),
        scratch_shapes=[
            pltpu.VMEM((C2, NPJ), jnp.float32),
            pltpu.VMEM((C2, C2), jnp.float32),
        ],
    )(XVp, XVp, Wm, Ws, M)

    R = B * NP
    idx_pad, val_pad = pl.pallas_call(
        _topk_body,
        out_shape=[
            jax.ShapeDtypeStruct((R, K), jnp.int32),
            jax.ShapeDtypeStruct((R, K), jnp.float32),
        ],
    )(diff.reshape(R, NPJ))

    index = idx_pad.reshape(B, NP, K)[:, :N, :].reshape(B, N * K)
    value = val_pad.reshape(B, NP, K)[:, :N, :].reshape(B, N * K, 1)
    return (index, value)


# speedup vs baseline: 8.1852x; 1.2319x over previous
import functools

import jax
import jax.numpy as jnp
from jax.experimental import pallas as pl
from jax.experimental.pallas import tpu as pltpu

N = 196
C = 384
C2 = 192
K = 8
NP = 224
NPJ = 256
BI = 112
NIB = NP // BI


def _dist_body(xv_blk, xv_full, wm, ws, m, diff_out, vt_s, a_s):
    b = pl.program_id(0)
    i = pl.program_id(1)

    @pl.when(jnp.logical_and(b == 0, i == 0))
    def _():
        a_s[...] = jax.lax.dot_general(
            m[...], m[...], (((0,), (0,)), ((), ())),
            preferred_element_type=jnp.float32)

    @pl.when(i == 0)
    def _():
        vt_s[...] = jnp.exp(jax.lax.dot_general(
            ws[...], xv_full[0], (((1,), (1,)), ((), ())),
            preferred_element_type=jnp.float32))

    ft = jnp.exp(-jax.lax.dot_general(
        wm[...], xv_blk[0], (((1,), (1,)), ((), ())),
        preferred_element_type=jnp.float32))

    ev = vt_s[...]
    a = a_s[...]

    s_cat = jnp.concatenate(
        [1.0 / (1.0 + ft[:, r:r + 1] * ev) for r in range(BI)],
        axis=1)
    t_cat = jax.lax.dot_general(
        a, s_cat, (((1,), (0,)), ((), ())),
        preferred_element_type=jnp.float32)
    p_cat = t_cat * s_cat
    rows = [jnp.sum(p_cat[:, r * NPJ:(r + 1) * NPJ], axis=0, keepdims=True)
            for r in range(BI)]
    d2b = jnp.maximum(jnp.concatenate(rows, axis=0), 0.0)
    diff = jax.nn.sigmoid(-jnp.sqrt(d2b))

    lane = jax.lax.broadcasted_iota(jnp.int32, (BI, NPJ), 1)
    diff_out[0] = jnp.where(lane < N, diff, -jnp.inf)


def _topk_body(diff_ref, idx_out, val_out):
    diff = diff_ref[...]
    R = diff.shape[0]
    lane = jax.lax.broadcasted_iota(jnp.int32, (R, NPJ), 1)
    vals, idxs = [], []
    for _ in range(K):
        mx = jnp.max(diff, axis=1, keepdims=True)
        amx = jnp.min(jnp.where(diff == mx, lane, NPJ), axis=1,
                      keepdims=True)
        vals.append(-mx)
        idxs.append(amx)
        diff = jnp.where(lane == amx, -jnp.inf, diff)
    val_out[...] = jnp.concatenate(vals, axis=1)
    idx_out[...] = jnp.concatenate(idxs, axis=1)


@jax.jit
def kernel(X, Wm, Ws, M):
    B = X.shape[0]
    XV = jnp.transpose(X, (0, 2, 3, 1)).reshape(B, N, C)
    XVp = jnp.pad(XV, ((0, 0), (0, NPJ - N), (0, 0)))

    diff = pl.pallas_call(
        _dist_body,
        grid=(B, NIB),
        in_specs=[
            pl.BlockSpec((1, BI, C), lambda b, i: (b, i, 0)),
            pl.BlockSpec((1, NPJ, C), lambda b, i: (b, 0, 0)),
            pl.BlockSpec((C2, C), lambda b, i: (0, 0)),
            pl.BlockSpec((C2, C), lambda b, i: (0, 0)),
            pl.BlockSpec((C2, C2), lambda b, i: (0, 0)),
        ],
        out_specs=pl.BlockSpec((1, BI, NPJ), lambda b, i: (b, i, 0)),
        out_shape=jax.ShapeDtypeStruct((B, NP, NPJ), jnp.float32),
        scratch_shapes=[
            pltpu.VMEM((C2, NPJ), jnp.float32),
            pltpu.VMEM((C2, C2), jnp.float32),
        ],
    )(XVp, XVp, Wm, Ws, M)

    R = B * NP
    idx_pad, val_pad = pl.pallas_call(
        _topk_body,
        out_shape=[
            jax.ShapeDtypeStruct((R, K), jnp.int32),
            jax.ShapeDtypeStruct((R, K), jnp.float32),
        ],
    )(diff.reshape(R, NPJ))

    index = idx_pad.reshape(B, NP, K)[:, :N, :].reshape(B, N * K)
    value = val_pad.reshape(B, NP, K)[:, :N, :].reshape(B, N * K, 1)
    return (index, value)


# speedup vs baseline: 9.8209x; 1.1998x over previous
import jax
import jax.numpy as jnp
from jax.experimental import pallas as pl
from jax.experimental.pallas import tpu as pltpu

N = 196
C = 384
C2 = 192
K = 8
NPJ = 256
NP = 224
NROW = 200
R = 2 * NP


def _srknn_body(xf, wm, ws, m, idx_out, val_out, diff_s):
    b = pl.program_id(0)
    nb = pl.num_programs(0)

    @pl.when(b == 0)
    def _():
        diff_s[...] = jnp.full((R, NPJ), -jnp.inf, dtype=jnp.float32)

    a = jax.lax.dot_general(
        m[...], m[...], (((0,), (0,)), ((), ())),
        preferred_element_type=jnp.float32)
    ev = jnp.exp(jax.lax.dot_general(
        ws[...], xf[0], (((1,), (0,)), ((), ())),
        preferred_element_type=jnp.float32))
    ft = jnp.exp(-jax.lax.dot_general(
        wm[...], xf[0], (((1,), (0,)), ((), ())),
        preferred_element_type=jnp.float32))

    s_cat = jnp.concatenate(
        [1.0 / (1.0 + ft[:, r:r + 1] * ev) for r in range(NROW)],
        axis=1)
    t_cat = jax.lax.dot_general(
        a, s_cat, (((1,), (0,)), ((), ())),
        preferred_element_type=jnp.float32)
    rows = [jnp.sum(t_cat[:, r * NPJ:(r + 1) * NPJ] *
                    s_cat[:, r * NPJ:(r + 1) * NPJ], axis=0, keepdims=True)
            for r in range(NROW)]
    d2b = jnp.maximum(jnp.concatenate(rows, axis=0), 0.0)
    diff = jax.nn.sigmoid(-jnp.sqrt(d2b))

    lane_b = jax.lax.broadcasted_iota(jnp.int32, (NROW, NPJ), 1)
    diff_s[pl.ds(b * NP, NROW), :] = jnp.where(lane_b < N, diff, -jnp.inf)

    @pl.when(b == nb - 1)
    def _():
        diffall = diff_s[...]
        lane = jax.lax.broadcasted_iota(jnp.int32, (R, NPJ), 1)
        vals, idxs = [], []
        for _ in range(K):
            mx = jnp.max(diffall, axis=1, keepdims=True)
            amx = jnp.min(jnp.where(diffall == mx, lane, NPJ), axis=1,
                          keepdims=True)
            vals.append(-mx)
            idxs.append(amx)
            diffall = jnp.where(lane == amx, -jnp.inf, diffall)
        val_out[...] = jnp.concatenate(vals, axis=1)
        idx_out[...] = jnp.concatenate(idxs, axis=1)


@jax.jit
def kernel(X, Wm, Ws, M):
    B = X.shape[0]
    Xf = X.reshape(B, C, N)
    Xfp = jnp.pad(Xf, ((0, 0), (0, 0), (0, NPJ - N)))

    idx_pad, val_pad = pl.pallas_call(
        _srknn_body,
        grid=(B,),
        in_specs=[
            pl.BlockSpec((1, C, NPJ), lambda b: (b, 0, 0)),
            pl.BlockSpec((C2, C), lambda b: (0, 0)),
            pl.BlockSpec((C2, C), lambda b: (0, 0)),
            pl.BlockSpec((C2, C2), lambda b: (0, 0)),
        ],
        out_specs=[
            pl.BlockSpec((R, K), lambda b: (0, 0)),
            pl.BlockSpec((R, K), lambda b: (0, 0)),
        ],
        out_shape=[
            jax.ShapeDtypeStruct((R, K), jnp.int32),
            jax.ShapeDtypeStruct((R, K), jnp.float32),
        ],
        scratch_shapes=[
            pltpu.VMEM((R, NPJ), jnp.float32),
        ],
    )(Xfp, Wm, Ws, M)

    index = idx_pad.reshape(B, NP, K)[:, :N, :].reshape(B, N * K)
    value = val_pad.reshape(B, NP, K)[:, :N, :].reshape(B, N * K, 1)
    return (index, value)
